# Initial kernel scaffold; baseline (speedup 1.0000x reference)
import jax
import jax.numpy as jnp
from jax.experimental import pallas as pl


def kernel(x, edge_index, edge_weight, W1, b1, W2, b2):
    raise NotImplementedError("write your pallas kernel here")


# speedup vs baseline: 6.0650x; 6.0650x over previous
import functools

import jax
import jax.numpy as jnp
from jax import lax
from jax.experimental import pallas as pl
from jax.experimental.pallas import tpu as pltpu
from jax.experimental.pallas import tpu_sc as plsc

N = 10000
E = 160000
D = 256
HALF = 128
NC = 2
NS = 16
L = 16
NP = 10240
NPS = NP // NS
CH = 80
EPS = E // NS
NECH = EPS // CH
NNCH = NPS // CH
RB = 512


def _lin_body(xa_ref, xb_ref, w_ref, b_ref, o_ref):
    x = jnp.concatenate([xa_ref[...].reshape(RB, HALF),
                         xb_ref[...].reshape(RB, HALF)], axis=1)
    w = w_ref[0]
    o = lax.dot_general(x, w, (((1,), (1,)), ((), ())),
                        preferred_element_type=jnp.float32)
    o_ref[0] = o + b_ref[0]


def _linear_halves_from_full(x, W, b):
    w3 = W.reshape(NC, HALF, D)
    b2 = b.reshape(NC, 1, HALF)
    grid = (NC, NP // RB)
    return pl.pallas_call(
        _lin_body,
        grid=grid,
        in_specs=[
            pl.BlockSpec((RB, HALF), lambda c, i: (i, 0)),
            pl.BlockSpec((RB, HALF), lambda c, i: (i, 1)),
            pl.BlockSpec((1, HALF, D), lambda c, i: (c, 0, 0)),
            pl.BlockSpec((1, 1, HALF), lambda c, i: (c, 0, 0)),
        ],
        out_specs=pl.BlockSpec((1, RB, HALF), lambda c, i: (c, i, 0)),
        out_shape=jax.ShapeDtypeStruct((NC, NP, HALF), jnp.float32),
    )(x, x, w3, b2)


def _linear_halves_from_halves(h, W, b):
    w3 = W.reshape(NC, HALF, D)
    b2 = b.reshape(NC, 1, HALF)
    grid = (NC, NP // RB)
    return pl.pallas_call(
        _lin_body,
        grid=grid,
        in_specs=[
            pl.BlockSpec((1, RB, HALF), lambda c, i: (0, i, 0)),
            pl.BlockSpec((1, RB, HALF), lambda c, i: (1, i, 0)),
            pl.BlockSpec((1, HALF, D), lambda c, i: (c, 0, 0)),
            pl.BlockSpec((1, 1, HALF), lambda c, i: (c, 0, 0)),
        ],
        out_specs=pl.BlockSpec((1, RB, HALF), lambda c, i: (c, i, 0)),
        out_shape=jax.ShapeDtypeStruct((NC, NP, HALF), jnp.float32),
    )(h, h, w3, b2)


def _rsqrt16(x):
    i = lax.bitcast_convert_type(x, jnp.int32)
    magic = jnp.full((L,), 0x5F3759DF, jnp.int32)
    y = lax.bitcast_convert_type(magic - lax.shift_right_logical(i, 1),
                                 jnp.float32)
    half = x * (-0.5)
    for _ in range(3):
        y = y * (half * y * y + 1.5)
    return y


def _splat(ref, i):
    return plsc.load_gather(ref, [jnp.full((L,), i, jnp.int32)])


def _make_sc_propagate(first_layer: bool):
    mesh = plsc.VectorSubcoreMesh(core_axis_name="c", subcore_axis_name="s",
                                  num_cores=NC)
    if first_layer:
        out_type = (jax.ShapeDtypeStruct((NC * NP, HALF), jnp.float32),
                    jax.ShapeDtypeStruct((NP,), jnp.float32))
    else:
        out_type = jax.ShapeDtypeStruct((NC * NP, HALF), jnp.float32)

    scratch = dict(
        epk_v=pltpu.VMEM((3, CH), jnp.int32),
        rowo_v=pltpu.VMEM((CH,), jnp.int32),
        dis_v=pltpu.VMEM((NP,), jnp.float32),
        rows_v=pltpu.VMEM((CH, HALF), jnp.float32),
        w_v=pltpu.VMEM((CH,), jnp.float32),
        t_v=pltpu.VMEM((NPS,), jnp.float32),
        ones_v=pltpu.VMEM((CH,), jnp.float32),
        acc_sh=pltpu.VMEM_SHARED((NP, HALF), jnp.float32),
        deg_sh=pltpu.VMEM_SHARED((NP,), jnp.float32),
        sem=pltpu.SemaphoreType.DMA,
    )

    def body(epk_hbm, hw_hbm, dis_hbm, h_out, dis_out,
             epk_v, rowo_v, dis_v, rows_v, w_v, t_v, ones_v,
             acc_sh, deg_sh, sem):
        c = lax.axis_index("c")
        s = lax.axis_index("s")
        nbase = s * NPS
        zero16 = jnp.zeros((L,), jnp.float32)
        one16 = jnp.ones((L,), jnp.float32)
        coff = jnp.full((L,), c * NP, jnp.int32)

        def fill(ref, n, vec):
            def st(i, _):
                ref[pl.ds(i * L, L)] = vec
                return 0
            lax.fori_loop(0, n // L, st, 0)

        if first_layer:
            fill(ones_v, CH, one16)
            fill(t_v, NPS, zero16)
            pltpu.sync_copy(t_v, deg_sh.at[pl.ds(nbase, NPS)])
            plsc.subcore_barrier()

            def deg_chunk(j, _):
                pltpu.sync_copy(epk_hbm.at[s, j], epk_v)
                pltpu.sync_copy(ones_v, deg_sh.at[epk_v.at[1]], add=True)
                return 0
            lax.fori_loop(0, NECH, deg_chunk, 0)
            plsc.subcore_barrier()

            pltpu.sync_copy(deg_sh.at[pl.ds(nbase, NPS)], t_v)

            def mk_dis(i, _):
                sl = pl.ds(i * L, L)
                t_v[sl] = _rsqrt16(t_v[sl] + 1.0)
                return 0
            lax.fori_loop(0, NPS // L, mk_dis, 0)
            pltpu.sync_copy(t_v, deg_sh.at[pl.ds(nbase, NPS)])

            @pl.when(c == 0)
            def _():
                pltpu.sync_copy(t_v, dis_out.at[pl.ds(nbase, NPS)])
            plsc.subcore_barrier()
            pltpu.sync_copy(deg_sh, dis_v)
        else:
            pltpu.sync_copy(dis_hbm, dis_v)

        def init_chunk(k, _):
            base = nbase + k * CH
            pltpu.sync_copy(hw_hbm.at[pl.ds(c * NP + base, CH)], rows_v)

            def init_e(e, _):
                dvec = _splat(dis_v, base + e)
                for g in range(HALF // L):
                    sl = pl.ds(g * L, L)
                    rows_v[e, sl] = rows_v[e, sl] * dvec
                return 0
            lax.fori_loop(0, CH, init_e, 0)
            pltpu.sync_copy(rows_v, acc_sh.at[pl.ds(base, CH)])
            return 0
        lax.fori_loop(0, NNCH, init_chunk, 0)
        plsc.subcore_barrier()

        def edge_chunk(j, _):
            pltpu.sync_copy(epk_hbm.at[s, j], epk_v)

            def mk_w(g, _):
                sl = pl.ds(g * L, L)
                r16 = epk_v[0, sl]
                rowo_v[sl] = r16 + coff
                d16 = plsc.load_gather(dis_v, [r16])
                e16 = lax.bitcast_convert_type(epk_v[2, sl], jnp.float32)
                w_v[sl] = d16 * e16
                return 0
            lax.fori_loop(0, CH // L, mk_w, 0)

            pltpu.async_copy(hw_hbm.at[rowo_v], rows_v, sem).wait()

            def scale_e(e, _):
                wvec = _splat(w_v, e)
                for g in range(HALF // L):
                    sl = pl.ds(g * L, L)
                    rows_v[e, sl] = rows_v[e, sl] * wvec
                return 0
            lax.fori_loop(0, CH, scale_e, 0)

            pltpu.sync_copy(rows_v, acc_sh.at[epk_v.at[1]], add=True)
            return 0
        lax.fori_loop(0, NECH, edge_chunk, 0)
        plsc.subcore_barrier()

        def flush_chunk(k, _):
            base = nbase + k * CH
            pltpu.sync_copy(acc_sh.at[pl.ds(base, CH)], rows_v)

            def flush_e(e, _):
                dvec = _splat(dis_v, base + e)
                for g in range(HALF // L):
                    sl = pl.ds(g * L, L)
                    v = rows_v[e, sl] * dvec
                    if first_layer:
                        v = jnp.maximum(v, 0.0)
                    else:
                        v = 1.0 / (1.0 + jnp.exp(-v))
                    rows_v[e, sl] = v
                return 0
            lax.fori_loop(0, CH, flush_e, 0)
            pltpu.sync_copy(rows_v, h_out.at[pl.ds(c * NP + base, CH)])
            return 0
        lax.fori_loop(0, NNCH, flush_chunk, 0)

    if first_layer:
        @functools.partial(
            pl.kernel, out_type=out_type, mesh=mesh, scratch_types=scratch,
            compiler_params=pltpu.CompilerParams(needs_layout_passes=False))
        def k1(epk, hw, h_out, dis_out, *, epk_v, rowo_v, dis_v, rows_v,
               w_v, t_v, ones_v, acc_sh, deg_sh, sem):
            body(epk, hw, None, h_out, dis_out,
                 epk_v, rowo_v, dis_v, rows_v, w_v, t_v, ones_v,
                 acc_sh, deg_sh, sem)
        return k1
    else:
        @functools.partial(
            pl.kernel, out_type=out_type, mesh=mesh, scratch_types=scratch,
            compiler_params=pltpu.CompilerParams(needs_layout_passes=False))
        def k2(epk, hw, dis_hbm, h_out, *, epk_v, rowo_v, dis_v, rows_v,
               w_v, t_v, ones_v, acc_sh, deg_sh, sem):
            body(epk, hw, dis_hbm, h_out, None,
                 epk_v, rowo_v, dis_v, rows_v, w_v, t_v, ones_v,
                 acc_sh, deg_sh, sem)
        return k2


_sc_layer1 = _make_sc_propagate(True)
_sc_layer2 = _make_sc_propagate(False)


@jax.jit
def kernel(x, edge_index, edge_weight, W1, b1, W2, b2):
    row = edge_index[0].reshape(NS, NECH, CH)
    col = edge_index[1].reshape(NS, NECH, CH)
    ewb = lax.bitcast_convert_type(
        edge_weight.reshape(NS, NECH, CH), jnp.int32)
    epk = jnp.stack([row, col, ewb], axis=2)

    xp = jnp.pad(x, ((0, NP - N), (0, 0)))
    hw1 = _linear_halves_from_full(xp, W1, b1).reshape(NC * NP, HALF)
    h1_flat, dis = _sc_layer1(epk, hw1)
    h1 = h1_flat.reshape(NC, NP, HALF)
    hw2 = _linear_halves_from_halves(h1, W2, b2).reshape(NC * NP, HALF)
    out_flat = _sc_layer2(epk, hw2, dis)
    return jnp.concatenate([out_flat[:N], out_flat[NP:NP + N]], axis=1)


# speedup vs baseline: 8.7738x; 1.4466x over previous
import functools

import jax
import jax.numpy as jnp
from jax import lax
from jax.experimental import pallas as pl
from jax.experimental.pallas import tpu as pltpu
from jax.experimental.pallas import tpu_sc as plsc

N = 10000
E = 160000
D = 256
HALF = 128
NC = 2
NS = 16
L = 16
NP = 10240
NPS = NP // NS
CH = 80
EPS = E // NS
NECH = EPS // CH
NNCH = NPS // CH
RB = 512


def _lin_body(xa_ref, xb_ref, w_ref, b_ref, o_ref):
    x = jnp.concatenate([xa_ref[...].reshape(RB, HALF),
                         xb_ref[...].reshape(RB, HALF)], axis=1)
    w = w_ref[0]
    o = lax.dot_general(x, w, (((1,), (1,)), ((), ())),
                        preferred_element_type=jnp.float32)
    o_ref[0] = o + b_ref[0]


def _linear_halves_from_full(x, W, b):
    w3 = W.reshape(NC, HALF, D)
    b2 = b.reshape(NC, 1, HALF)
    grid = (NC, NP // RB)
    return pl.pallas_call(
        _lin_body,
        grid=grid,
        in_specs=[
            pl.BlockSpec((RB, HALF), lambda c, i: (i, 0)),
            pl.BlockSpec((RB, HALF), lambda c, i: (i, 1)),
            pl.BlockSpec((1, HALF, D), lambda c, i: (c, 0, 0)),
            pl.BlockSpec((1, 1, HALF), lambda c, i: (c, 0, 0)),
        ],
        out_specs=pl.BlockSpec((1, RB, HALF), lambda c, i: (c, i, 0)),
        out_shape=jax.ShapeDtypeStruct((NC, NP, HALF), jnp.float32),
    )(x, x, w3, b2)


def _linear_halves_from_halves(h, W, b):
    w3 = W.reshape(NC, HALF, D)
    b2 = b.reshape(NC, 1, HALF)
    grid = (NC, NP // RB)
    return pl.pallas_call(
        _lin_body,
        grid=grid,
        in_specs=[
            pl.BlockSpec((1, RB, HALF), lambda c, i: (0, i, 0)),
            pl.BlockSpec((1, RB, HALF), lambda c, i: (1, i, 0)),
            pl.BlockSpec((1, HALF, D), lambda c, i: (c, 0, 0)),
            pl.BlockSpec((1, 1, HALF), lambda c, i: (c, 0, 0)),
        ],
        out_specs=pl.BlockSpec((1, RB, HALF), lambda c, i: (c, i, 0)),
        out_shape=jax.ShapeDtypeStruct((NC, NP, HALF), jnp.float32),
    )(h, h, w3, b2)


def _rsqrt16(x):
    i = lax.bitcast_convert_type(x, jnp.int32)
    magic = jnp.full((L,), 0x5F3759DF, jnp.int32)
    y = lax.bitcast_convert_type(magic - lax.shift_right_logical(i, 1),
                                 jnp.float32)
    half = x * (-0.5)
    for _ in range(3):
        y = y * (half * y * y + 1.5)
    return y


def _splat(ref, i):
    return plsc.load_gather(ref, [jnp.full((L,), i, jnp.int32)])


def _make_sc_propagate(first_layer: bool):
    mesh = plsc.VectorSubcoreMesh(core_axis_name="c", subcore_axis_name="s",
                                  num_cores=NC)
    if first_layer:
        out_type = (jax.ShapeDtypeStruct((NC * NP, HALF), jnp.float32),
                    jax.ShapeDtypeStruct((NP,), jnp.float32))
    else:
        out_type = jax.ShapeDtypeStruct((NC * NP, HALF), jnp.float32)

    scratch = dict(
        epk_v=[pltpu.VMEM((3, CH), jnp.int32)] * 2,
        rowo_v=[pltpu.VMEM((CH,), jnp.int32)] * 2,
        col_v=[pltpu.VMEM((CH,), jnp.int32)] * 2,
        w_v=[pltpu.VMEM((CH,), jnp.float32)] * 2,
        rows_v=[pltpu.VMEM((CH, HALF), jnp.float32)] * 2,
        dis_v=pltpu.VMEM((NP,), jnp.float32),
        t_v=pltpu.VMEM((NPS,), jnp.float32),
        ones_v=pltpu.VMEM((CH,), jnp.float32),
        acc_sh=pltpu.VMEM_SHARED((NP, HALF), jnp.float32),
        deg_sh=pltpu.VMEM_SHARED((NP,), jnp.float32),
        sem_e=pltpu.SemaphoreType.DMA,
        sem_g=[pltpu.SemaphoreType.DMA] * 2,
        sem_s=pltpu.SemaphoreType.DMA,
    )

    def body(epk_hbm, hw_hbm, dis_hbm, h_out, dis_out,
             epk_v, rowo_v, col_v, w_v, rows_v, dis_v, t_v, ones_v,
             acc_sh, deg_sh, sem_e, sem_g, sem_s):
        c = lax.axis_index("c")
        s = lax.axis_index("s")
        nbase = s * NPS
        zero16 = jnp.zeros((L,), jnp.float32)
        one16 = jnp.ones((L,), jnp.float32)
        coff = jnp.full((L,), c * NP, jnp.int32)

        def fill(ref, n, vec):
            def st(i, _):
                ref[pl.ds(i * L, L)] = vec
                return 0
            lax.fori_loop(0, n // L, st, 0)

        if first_layer:
            fill(ones_v, CH, one16)
            fill(t_v, NPS, zero16)
            pltpu.sync_copy(t_v, deg_sh.at[pl.ds(nbase, NPS)])
            plsc.subcore_barrier()

            pltpu.sync_copy(epk_hbm.at[s, 0], epk_v[0])
            pltpu.async_copy(epk_hbm.at[s, 1], epk_v[1], sem_e)

            def deg_iter(j, b):
                pltpu.sync_copy(ones_v, deg_sh.at[epk_v[b].at[1]], add=True)
                pltpu.make_async_copy(epk_hbm.at[s, j + 1], epk_v[1 - b],
                                      sem_e).wait()
                pltpu.async_copy(epk_hbm.at[s, j + 2], epk_v[b], sem_e)

            def deg_pair(t, _):
                deg_iter(2 * t, 0)
                deg_iter(2 * t + 1, 1)
                return 0
            lax.fori_loop(0, (NECH - 1) // 2, deg_pair, 0)
            pltpu.sync_copy(ones_v, deg_sh.at[epk_v[0].at[1]], add=True)
            pltpu.make_async_copy(epk_hbm.at[s, NECH], epk_v[1],
                                  sem_e).wait()
            plsc.subcore_barrier()

            pltpu.sync_copy(deg_sh.at[pl.ds(nbase, NPS)], t_v)

            def mk_dis(i, _):
                sl = pl.ds(i * L, L)
                t_v[sl] = _rsqrt16(t_v[sl] + 1.0)
                return 0
            lax.fori_loop(0, NPS // L, mk_dis, 0)
            pltpu.sync_copy(t_v, deg_sh.at[pl.ds(nbase, NPS)])

            @pl.when(c == 0)
            def _():
                pltpu.sync_copy(t_v, dis_out.at[pl.ds(nbase, NPS)])
            plsc.subcore_barrier()
            pltpu.sync_copy(deg_sh, dis_v)
        else:
            pltpu.sync_copy(dis_hbm, dis_v)

        def init_chunk(k, _):
            base = nbase + k * CH
            buf = rows_v[0]
            pltpu.sync_copy(hw_hbm.at[pl.ds(c * NP + base, CH)], buf)

            def init_e(e, _):
                dvec = _splat(dis_v, base + e)
                for g in range(HALF // L):
                    sl = pl.ds(g * L, L)
                    buf[e, sl] = buf[e, sl] * dvec
                return 0
            lax.fori_loop(0, CH, init_e, 0)
            pltpu.sync_copy(buf, acc_sh.at[pl.ds(base, CH)])
            return 0
        lax.fori_loop(0, NNCH, init_chunk, 0)
        plsc.subcore_barrier()

        def prep(b):
            def grp(g, _):
                sl = pl.ds(g * L, L)
                r16 = epk_v[b][0, sl]
                rowo_v[b][sl] = r16 + coff
                d16 = plsc.load_gather(dis_v, [r16])
                e16 = lax.bitcast_convert_type(epk_v[b][2, sl], jnp.float32)
                w_v[b][sl] = d16 * e16
                col_v[b][sl] = epk_v[b][1, sl]
                return 0
            lax.fori_loop(0, CH // L, grp, 0)

        def issue_gather(b):
            pltpu.async_copy(hw_hbm.at[rowo_v[b]], rows_v[b], sem_g[b])

        def wait_gather(b):
            pltpu.make_async_copy(hw_hbm.at[rowo_v[b]], rows_v[b],
                                  sem_g[b]).wait()

        def issue_scatter(b):
            pltpu.async_copy(rows_v[b], acc_sh.at[col_v[b]], sem_s,
                             add=True)

        def wait_scatter(b):
            pltpu.make_async_copy(rows_v[b], acc_sh.at[col_v[b]],
                                  sem_s).wait()

        def scale(b):
            def scale_e(e, _):
                wvec = _splat(w_v[b], e)
                for g in range(HALF // L):
                    sl = pl.ds(g * L, L)
                    rows_v[b][e, sl] = rows_v[b][e, sl] * wvec
                return 0
            lax.fori_loop(0, CH, scale_e, 0)

        def edge_iter(j, b, first):
            pltpu.make_async_copy(epk_hbm.at[s, j + 1], epk_v[1 - b],
                                  sem_e).wait()
            if not first:
                wait_scatter(1 - b)
            prep(1 - b)
            issue_gather(1 - b)
            pltpu.async_copy(epk_hbm.at[s, j + 2], epk_v[b], sem_e)
            wait_gather(b)
            scale(b)
            issue_scatter(b)

        pltpu.sync_copy(epk_hbm.at[s, 0], epk_v[0])
        prep(0)
        issue_gather(0)
        pltpu.async_copy(epk_hbm.at[s, 1], epk_v[1], sem_e)
        edge_iter(0, 0, True)

        def edge_pair(t, _):
            edge_iter(2 * t + 1, 1, False)
            edge_iter(2 * t + 2, 0, False)
            return 0
        lax.fori_loop(0, (NECH - 1) // 2, edge_pair, 0)
        wait_scatter(0)
        wait_gather(1)
        pltpu.make_async_copy(epk_hbm.at[s, NECH + 1], epk_v[0],
                              sem_e).wait()
        plsc.subcore_barrier()

        def flush_chunk(k, _):
            base = nbase + k * CH
            buf = rows_v[0]
            pltpu.sync_copy(acc_sh.at[pl.ds(base, CH)], buf)

            def flush_e(e, _):
                dvec = _splat(dis_v, base + e)
                for g in range(HALF // L):
                    sl = pl.ds(g * L, L)
                    v = buf[e, sl] * dvec
                    if first_layer:
                        v = jnp.maximum(v, 0.0)
                    else:
                        v = 1.0 / (1.0 + jnp.exp(-v))
                    buf[e, sl] = v
                return 0
            lax.fori_loop(0, CH, flush_e, 0)
            pltpu.sync_copy(buf, h_out.at[pl.ds(c * NP + base, CH)])
            return 0
        lax.fori_loop(0, NNCH, flush_chunk, 0)

    if first_layer:
        @functools.partial(
            pl.kernel, out_type=out_type, mesh=mesh, scratch_types=scratch,
            compiler_params=pltpu.CompilerParams(needs_layout_passes=False))
        def k1(epk, hw, h_out, dis_out, *, epk_v, rowo_v, col_v, w_v,
               rows_v, dis_v, t_v, ones_v, acc_sh, deg_sh, sem_e, sem_g,
               sem_s):
            body(epk, hw, None, h_out, dis_out,
                 epk_v, rowo_v, col_v, w_v, rows_v, dis_v, t_v, ones_v,
                 acc_sh, deg_sh, sem_e, sem_g, sem_s)
        return k1
    else:
        @functools.partial(
            pl.kernel, out_type=out_type, mesh=mesh, scratch_types=scratch,
            compiler_params=pltpu.CompilerParams(needs_layout_passes=False))
        def k2(epk, hw, dis_hbm, h_out, *, epk_v, rowo_v, col_v, w_v,
               rows_v, dis_v, t_v, ones_v, acc_sh, deg_sh, sem_e, sem_g,
               sem_s):
            body(epk, hw, dis_hbm, h_out, None,
                 epk_v, rowo_v, col_v, w_v, rows_v, dis_v, t_v, ones_v,
                 acc_sh, deg_sh, sem_e, sem_g, sem_s)
        return k2


_sc_layer1 = _make_sc_propagate(True)
_sc_layer2 = _make_sc_propagate(False)


@jax.jit
def kernel(x, edge_index, edge_weight, W1, b1, W2, b2):
    row = edge_index[0].reshape(NS, NECH, CH)
    col = edge_index[1].reshape(NS, NECH, CH)
    ewb = lax.bitcast_convert_type(
        edge_weight.reshape(NS, NECH, CH), jnp.int32)
    epk = jnp.stack([row, col, ewb], axis=2)
    epk = jnp.pad(epk, ((0, 0), (0, 2), (0, 0), (0, 0)))

    xp = jnp.pad(x, ((0, NP - N), (0, 0)))
    hw1 = _linear_halves_from_full(xp, W1, b1).reshape(NC * NP, HALF)
    h1_flat, dis = _sc_layer1(epk, hw1)
    h1 = h1_flat.reshape(NC, NP, HALF)
    hw2 = _linear_halves_from_halves(h1, W2, b2).reshape(NC * NP, HALF)
    out_flat = _sc_layer2(epk, hw2, dis)
    return jnp.concatenate([out_flat[:N], out_flat[NP:NP + N]], axis=1)


# speedup vs baseline: 10.1485x; 1.1567x over previous
import functools

import jax
import jax.numpy as jnp
from jax import lax
from jax.experimental import pallas as pl
from jax.experimental.pallas import tpu as pltpu
from jax.experimental.pallas import tpu_sc as plsc

N = 10000
E = 160000
D = 256
HALF = 128
NC = 2
NS = 16
L = 16
NP = 10240
NPS = NP // NS
CH = 80
EPS = E // NS
NECH = EPS // CH
NNCH = NPS // CH
RB = 512


def _lin_body(xa_ref, xb_ref, w_ref, b_ref, o_ref):
    x = jnp.concatenate([xa_ref[...].reshape(RB, HALF),
                         xb_ref[...].reshape(RB, HALF)], axis=1)
    w = w_ref[0]
    o = lax.dot_general(x, w, (((1,), (1,)), ((), ())),
                        preferred_element_type=jnp.float32)
    o_ref[0] = o + b_ref[0]


def _linear_halves_from_full(x, W, b):
    w3 = W.reshape(NC, HALF, D)
    b2 = b.reshape(NC, 1, HALF)
    grid = (NC, NP // RB)
    return pl.pallas_call(
        _lin_body,
        grid=grid,
        in_specs=[
            pl.BlockSpec((RB, HALF), lambda c, i: (i, 0)),
            pl.BlockSpec((RB, HALF), lambda c, i: (i, 1)),
            pl.BlockSpec((1, HALF, D), lambda c, i: (c, 0, 0)),
            pl.BlockSpec((1, 1, HALF), lambda c, i: (c, 0, 0)),
        ],
        out_specs=pl.BlockSpec((1, RB, HALF), lambda c, i: (c, i, 0)),
        out_shape=jax.ShapeDtypeStruct((NC, NP, HALF), jnp.float32),
    )(x, x, w3, b2)


def _linear_halves_from_halves(h, W, b):
    w3 = W.reshape(NC, HALF, D)
    b2 = b.reshape(NC, 1, HALF)
    grid = (NC, NP // RB)
    return pl.pallas_call(
        _lin_body,
        grid=grid,
        in_specs=[
            pl.BlockSpec((1, RB, HALF), lambda c, i: (0, i, 0)),
            pl.BlockSpec((1, RB, HALF), lambda c, i: (1, i, 0)),
            pl.BlockSpec((1, HALF, D), lambda c, i: (c, 0, 0)),
            pl.BlockSpec((1, 1, HALF), lambda c, i: (c, 0, 0)),
        ],
        out_specs=pl.BlockSpec((1, RB, HALF), lambda c, i: (c, i, 0)),
        out_shape=jax.ShapeDtypeStruct((NC, NP, HALF), jnp.float32),
    )(h, h, w3, b2)


def _rsqrt16(x):
    i = lax.bitcast_convert_type(x, jnp.int32)
    magic = jnp.full((L,), 0x5F3759DF, jnp.int32)
    y = lax.bitcast_convert_type(magic - lax.shift_right_logical(i, 1),
                                 jnp.float32)
    half = x * (-0.5)
    for _ in range(3):
        y = y * (half * y * y + 1.5)
    return y


def _splat(ref, i):
    return plsc.load_gather(ref, [jnp.full((L,), i, jnp.int32)])


def _make_sc_propagate(first_layer: bool):
    mesh = plsc.VectorSubcoreMesh(core_axis_name="c", subcore_axis_name="s",
                                  num_cores=NC)
    if first_layer:
        out_type = (jax.ShapeDtypeStruct((NC * NP, HALF), jnp.float32),
                    jax.ShapeDtypeStruct((NP,), jnp.float32))
    else:
        out_type = jax.ShapeDtypeStruct((NC * NP, HALF), jnp.float32)

    scratch = dict(
        epk_v=[pltpu.VMEM((3, CH), jnp.int32)] * 2,
        rowo_v=[pltpu.VMEM((CH,), jnp.int32)] * 2,
        col_v=[pltpu.VMEM((CH,), jnp.int32)] * 2,
        w_v=[pltpu.VMEM((CH,), jnp.float32)] * 2,
        rows_v=[pltpu.VMEM((CH, HALF), jnp.float32)] * 2,
        dis_v=pltpu.VMEM((NP,), jnp.float32),
        t_v=pltpu.VMEM((NPS,), jnp.float32),
        ones_v=pltpu.VMEM((CH,), jnp.float32),
        acc_sh=pltpu.VMEM_SHARED((NP, HALF), jnp.float32),
        deg_sh=pltpu.VMEM_SHARED((NP,), jnp.float32),
        sem_e=pltpu.SemaphoreType.DMA,
        sem_g=[pltpu.SemaphoreType.DMA] * 2,
        sem_s=pltpu.SemaphoreType.DMA,
    )

    def body(epk_hbm, hw_hbm, dis_hbm, h_out, dis_out,
             epk_v, rowo_v, col_v, w_v, rows_v, dis_v, t_v, ones_v,
             acc_sh, deg_sh, sem_e, sem_g, sem_s):
        c = lax.axis_index("c")
        s = lax.axis_index("s")
        nbase = s * NPS
        zero16 = jnp.zeros((L,), jnp.float32)
        one16 = jnp.ones((L,), jnp.float32)
        coff = jnp.full((L,), c * NP, jnp.int32)

        def fill(ref, n, vec):
            def st(i, _):
                ref[pl.ds(i * L, L)] = vec
                return 0
            lax.fori_loop(0, n // L, st, 0)

        if first_layer:
            fill(ones_v, CH, one16)
            fill(t_v, NPS, zero16)
            pltpu.sync_copy(t_v, deg_sh.at[pl.ds(nbase, NPS)])
            plsc.subcore_barrier()

            pltpu.sync_copy(epk_hbm.at[s, 0], epk_v[0])
            pltpu.async_copy(epk_hbm.at[s, 1], epk_v[1], sem_e)

            def deg_iter(j, b):
                pltpu.sync_copy(ones_v, deg_sh.at[epk_v[b].at[1]], add=True)
                pltpu.make_async_copy(epk_hbm.at[s, j + 1], epk_v[1 - b],
                                      sem_e).wait()
                pltpu.async_copy(epk_hbm.at[s, j + 2], epk_v[b], sem_e)

            def deg_pair(t, _):
                deg_iter(2 * t, 0)
                deg_iter(2 * t + 1, 1)
                return 0
            lax.fori_loop(0, (NECH - 1) // 2, deg_pair, 0)
            pltpu.sync_copy(ones_v, deg_sh.at[epk_v[0].at[1]], add=True)
            pltpu.make_async_copy(epk_hbm.at[s, NECH], epk_v[1],
                                  sem_e).wait()
            plsc.subcore_barrier()

            pltpu.sync_copy(deg_sh.at[pl.ds(nbase, NPS)], t_v)

            def mk_dis(i, _):
                sl = pl.ds(i * L, L)
                t_v[sl] = _rsqrt16(t_v[sl] + 1.0)
                return 0
            lax.fori_loop(0, NPS // L, mk_dis, 0)
            pltpu.sync_copy(t_v, deg_sh.at[pl.ds(nbase, NPS)])

            @pl.when(c == 0)
            def _():
                pltpu.sync_copy(t_v, dis_out.at[pl.ds(nbase, NPS)])
            plsc.subcore_barrier()
            pltpu.sync_copy(deg_sh, dis_v)
        else:
            pltpu.sync_copy(dis_hbm, dis_v)

        def init_chunk(k, _):
            base = nbase + k * CH
            buf = rows_v[0]
            pltpu.sync_copy(hw_hbm.at[pl.ds(c * NP + base, CH)], buf)

            @plsc.parallel_loop(0, CH, unroll=4)
            def init_e(e):
                dvec = _splat(dis_v, base + e)
                for g in range(HALF // L):
                    sl = pl.ds(g * L, L)
                    buf[e, sl] = buf[e, sl] * dvec
            pltpu.sync_copy(buf, acc_sh.at[pl.ds(base, CH)])
            return 0
        lax.fori_loop(0, NNCH, init_chunk, 0)
        plsc.subcore_barrier()

        def prep(b):
            @plsc.parallel_loop(0, CH // L, unroll=5)
            def grp(g):
                sl = pl.ds(g * L, L)
                r16 = epk_v[b][0, sl]
                rowo_v[b][sl] = r16 + coff
                d16 = plsc.load_gather(dis_v, [r16])
                e16 = lax.bitcast_convert_type(epk_v[b][2, sl], jnp.float32)
                w_v[b][sl] = d16 * e16
                col_v[b][sl] = epk_v[b][1, sl]

        def issue_gather(b):
            pltpu.async_copy(hw_hbm.at[rowo_v[b]], rows_v[b], sem_g[b])

        def wait_gather(b):
            pltpu.make_async_copy(hw_hbm.at[rowo_v[b]], rows_v[b],
                                  sem_g[b]).wait()

        def issue_scatter(b):
            pltpu.async_copy(rows_v[b], acc_sh.at[col_v[b]], sem_s,
                             add=True)

        def wait_scatter(b):
            pltpu.make_async_copy(rows_v[b], acc_sh.at[col_v[b]],
                                  sem_s).wait()

        def scale(b):
            @plsc.parallel_loop(0, CH, unroll=4)
            def scale_e(e):
                wvec = _splat(w_v[b], e)
                for g in range(HALF // L):
                    sl = pl.ds(g * L, L)
                    rows_v[b][e, sl] = rows_v[b][e, sl] * wvec

        def edge_iter(j, b, first):
            pltpu.make_async_copy(epk_hbm.at[s, j + 1], epk_v[1 - b],
                                  sem_e).wait()
            if not first:
                wait_scatter(1 - b)
            prep(1 - b)
            issue_gather(1 - b)
            pltpu.async_copy(epk_hbm.at[s, j + 2], epk_v[b], sem_e)
            wait_gather(b)
            scale(b)
            issue_scatter(b)

        pltpu.sync_copy(epk_hbm.at[s, 0], epk_v[0])
        prep(0)
        issue_gather(0)
        pltpu.async_copy(epk_hbm.at[s, 1], epk_v[1], sem_e)
        edge_iter(0, 0, True)

        def edge_pair(t, _):
            edge_iter(2 * t + 1, 1, False)
            edge_iter(2 * t + 2, 0, False)
            return 0
        lax.fori_loop(0, (NECH - 1) // 2, edge_pair, 0)
        wait_scatter(0)
        wait_gather(1)
        pltpu.make_async_copy(epk_hbm.at[s, NECH + 1], epk_v[0],
                              sem_e).wait()
        plsc.subcore_barrier()

        def flush_chunk(k, _):
            base = nbase + k * CH
            buf = rows_v[0]
            pltpu.sync_copy(acc_sh.at[pl.ds(base, CH)], buf)

            @plsc.parallel_loop(0, CH, unroll=4)
            def flush_e(e):
                dvec = _splat(dis_v, base + e)
                for g in range(HALF // L):
                    sl = pl.ds(g * L, L)
                    v = buf[e, sl] * dvec
                    if first_layer:
                        v = jnp.maximum(v, 0.0)
                    else:
                        v = 1.0 / (1.0 + jnp.exp(-v))
                    buf[e, sl] = v
            pltpu.sync_copy(buf, h_out.at[pl.ds(c * NP + base, CH)])
            return 0
        lax.fori_loop(0, NNCH, flush_chunk, 0)

    if first_layer:
        @functools.partial(
            pl.kernel, out_type=out_type, mesh=mesh, scratch_types=scratch,
            compiler_params=pltpu.CompilerParams(needs_layout_passes=False))
        def k1(epk, hw, h_out, dis_out, *, epk_v, rowo_v, col_v, w_v,
               rows_v, dis_v, t_v, ones_v, acc_sh, deg_sh, sem_e, sem_g,
               sem_s):
            body(epk, hw, None, h_out, dis_out,
                 epk_v, rowo_v, col_v, w_v, rows_v, dis_v, t_v, ones_v,
                 acc_sh, deg_sh, sem_e, sem_g, sem_s)
        return k1
    else:
        @functools.partial(
            pl.kernel, out_type=out_type, mesh=mesh, scratch_types=scratch,
            compiler_params=pltpu.CompilerParams(needs_layout_passes=False))
        def k2(epk, hw, dis_hbm, h_out, *, epk_v, rowo_v, col_v, w_v,
               rows_v, dis_v, t_v, ones_v, acc_sh, deg_sh, sem_e, sem_g,
               sem_s):
            body(epk, hw, dis_hbm, h_out, None,
                 epk_v, rowo_v, col_v, w_v, rows_v, dis_v, t_v, ones_v,
                 acc_sh, deg_sh, sem_e, sem_g, sem_s)
        return k2


_sc_layer1 = _make_sc_propagate(True)
_sc_layer2 = _make_sc_propagate(False)


@jax.jit
def kernel(x, edge_index, edge_weight, W1, b1, W2, b2):
    row = edge_index[0].reshape(NS, NECH, CH)
    col = edge_index[1].reshape(NS, NECH, CH)
    ewb = lax.bitcast_convert_type(
        edge_weight.reshape(NS, NECH, CH), jnp.int32)
    epk = jnp.stack([row, col, ewb], axis=2)
    epk = jnp.pad(epk, ((0, 0), (0, 2), (0, 0), (0, 0)))

    xp = jnp.pad(x, ((0, NP - N), (0, 0)))
    hw1 = _linear_halves_from_full(xp, W1, b1).reshape(NC * NP, HALF)
    h1_flat, dis = _sc_layer1(epk, hw1)
    h1 = h1_flat.reshape(NC, NP, HALF)
    hw2 = _linear_halves_from_halves(h1, W2, b2).reshape(NC * NP, HALF)
    out_flat = _sc_layer2(epk, hw2, dis)
    return jnp.concatenate([out_flat[:N], out_flat[NP:NP + N]], axis=1)


# speedup vs baseline: 10.7334x; 1.0576x over previous
import functools

import jax
import jax.numpy as jnp
from jax import lax
from jax.experimental import pallas as pl
from jax.experimental.pallas import tpu as pltpu
from jax.experimental.pallas import tpu_sc as plsc

N = 10000
E = 160000
D = 256
HALF = 128
NC = 2
NS = 16
L = 16
NP = 10240
NPS = NP // NS
CH = 80
EPS = E // NS
NECH = EPS // CH
NNCH = NPS // CH
RB = 512


def _lin_body(xa_ref, xb_ref, w_ref, b_ref, o_ref):
    x = jnp.concatenate([xa_ref[...].reshape(RB, HALF),
                         xb_ref[...].reshape(RB, HALF)], axis=1)
    w = w_ref[0]
    o = lax.dot_general(x, w, (((1,), (1,)), ((), ())),
                        preferred_element_type=jnp.float32)
    o_ref[0] = o + b_ref[0]


def _linear_halves_from_full(x, W, b):
    w3 = W.reshape(NC, HALF, D)
    b2 = b.reshape(NC, 1, HALF)
    grid = (NC, NP // RB)
    return pl.pallas_call(
        _lin_body,
        grid=grid,
        in_specs=[
            pl.BlockSpec((RB, HALF), lambda c, i: (i, 0)),
            pl.BlockSpec((RB, HALF), lambda c, i: (i, 1)),
            pl.BlockSpec((1, HALF, D), lambda c, i: (c, 0, 0)),
            pl.BlockSpec((1, 1, HALF), lambda c, i: (c, 0, 0)),
        ],
        out_specs=pl.BlockSpec((1, RB, HALF), lambda c, i: (c, i, 0)),
        out_shape=jax.ShapeDtypeStruct((NC, NP, HALF), jnp.float32),
    )(x, x, w3, b2)


def _linear_halves_from_halves(h, W, b):
    w3 = W.reshape(NC, HALF, D)
    b2 = b.reshape(NC, 1, HALF)
    grid = (NC, NP // RB)
    return pl.pallas_call(
        _lin_body,
        grid=grid,
        in_specs=[
            pl.BlockSpec((1, RB, HALF), lambda c, i: (0, i, 0)),
            pl.BlockSpec((1, RB, HALF), lambda c, i: (1, i, 0)),
            pl.BlockSpec((1, HALF, D), lambda c, i: (c, 0, 0)),
            pl.BlockSpec((1, 1, HALF), lambda c, i: (c, 0, 0)),
        ],
        out_specs=pl.BlockSpec((1, RB, HALF), lambda c, i: (c, i, 0)),
        out_shape=jax.ShapeDtypeStruct((NC, NP, HALF), jnp.float32),
    )(h, h, w3, b2)


def _rsqrt16(x):
    i = lax.bitcast_convert_type(x, jnp.int32)
    magic = jnp.full((L,), 0x5F3759DF, jnp.int32)
    y = lax.bitcast_convert_type(magic - lax.shift_right_logical(i, 1),
                                 jnp.float32)
    half = x * (-0.5)
    for _ in range(3):
        y = y * (half * y * y + 1.5)
    return y


def _splat(ref, i):
    return plsc.load_gather(ref, [jnp.full((L,), i, jnp.int32)])


def _sc_degree():
    mesh = plsc.VectorSubcoreMesh(core_axis_name="c", subcore_axis_name="s",
                                  num_cores=NC)
    scratch = dict(
        epk_v=[pltpu.VMEM((3, CH), jnp.int32)] * 2,
        t_v=pltpu.VMEM((NPS,), jnp.float32),
        ones_v=pltpu.VMEM((CH,), jnp.float32),
        deg_sh=pltpu.VMEM_SHARED((NP,), jnp.float32),
        sem_e=pltpu.SemaphoreType.DMA,
    )

    @functools.partial(
        pl.kernel, out_type=jax.ShapeDtypeStruct((NP,), jnp.float32),
        mesh=mesh, scratch_types=scratch,
        compiler_params=pltpu.CompilerParams(needs_layout_passes=False))
    def kdeg(epk_hbm, dis_out, *, epk_v, t_v, ones_v, deg_sh, sem_e):
        c = lax.axis_index("c")
        s = lax.axis_index("s")
        nbase = s * NPS
        zero16 = jnp.zeros((L,), jnp.float32)
        one16 = jnp.ones((L,), jnp.float32)

        def fill(ref, n, vec):
            def st(i, _):
                ref[pl.ds(i * L, L)] = vec
                return 0
            lax.fori_loop(0, n // L, st, 0)

        fill(ones_v, CH, one16)
        fill(t_v, NPS, zero16)
        pltpu.sync_copy(t_v, deg_sh.at[pl.ds(nbase, NPS)])
        plsc.subcore_barrier()

        pltpu.sync_copy(epk_hbm.at[s, 0], epk_v[0])
        pltpu.async_copy(epk_hbm.at[s, 1], epk_v[1], sem_e)

        def deg_iter(j, b):
            pltpu.sync_copy(ones_v, deg_sh.at[epk_v[b].at[1]], add=True)
            pltpu.make_async_copy(epk_hbm.at[s, j + 1], epk_v[1 - b],
                                  sem_e).wait()
            pltpu.async_copy(epk_hbm.at[s, j + 2], epk_v[b], sem_e)

        def deg_pair(t, _):
            deg_iter(2 * t, 0)
            deg_iter(2 * t + 1, 1)
            return 0
        lax.fori_loop(0, (NECH - 1) // 2, deg_pair, 0)
        pltpu.sync_copy(ones_v, deg_sh.at[epk_v[0].at[1]], add=True)
        pltpu.make_async_copy(epk_hbm.at[s, NECH], epk_v[1], sem_e).wait()
        plsc.subcore_barrier()

        pltpu.sync_copy(deg_sh.at[pl.ds(nbase, NPS)], t_v)

        def mk_dis(i, _):
            sl = pl.ds(i * L, L)
            t_v[sl] = _rsqrt16(t_v[sl] + 1.0)
            return 0
        lax.fori_loop(0, NPS // L, mk_dis, 0)

        @pl.when(c == 0)
        def _():
            pltpu.sync_copy(t_v, dis_out.at[pl.ds(nbase, NPS)])

    return kdeg


def _make_sc_propagate(final: bool):
    mesh = plsc.VectorSubcoreMesh(core_axis_name="c", subcore_axis_name="s",
                                  num_cores=NC)
    if final:
        out_type = jax.ShapeDtypeStruct((N, D), jnp.float32)
    else:
        out_type = jax.ShapeDtypeStruct((NC * NP, HALF), jnp.float32)

    scratch = dict(
        epk_v=[pltpu.VMEM((3, CH), jnp.int32)] * 2,
        rowo_v=[pltpu.VMEM((CH,), jnp.int32)] * 2,
        col_v=[pltpu.VMEM((CH,), jnp.int32)] * 2,
        w_v=[pltpu.VMEM((CH,), jnp.float32)] * 2,
        rows_v=[pltpu.VMEM((CH, HALF), jnp.float32)] * 2,
        dis_v=pltpu.VMEM((NP,), jnp.float32),
        acc_sh=pltpu.VMEM_SHARED((NP, HALF), jnp.float32),
        sem_e=pltpu.SemaphoreType.DMA,
        sem_g=[pltpu.SemaphoreType.DMA] * 2,
        sem_s=pltpu.SemaphoreType.DMA,
    )

    @functools.partial(
        pl.kernel, out_type=out_type, mesh=mesh, scratch_types=scratch,
        compiler_params=pltpu.CompilerParams(needs_layout_passes=False))
    def kprop(epk_hbm, hw_hbm, dis_hbm, h_out, *, epk_v, rowo_v, col_v,
              w_v, rows_v, dis_v, acc_sh, sem_e, sem_g, sem_s):
        c = lax.axis_index("c")
        s = lax.axis_index("s")
        nbase = s * NPS
        coff = jnp.full((L,), c * NP, jnp.int32)

        pltpu.sync_copy(dis_hbm, dis_v)

        def init_chunk(k, _):
            base = nbase + k * CH
            buf = rows_v[0]
            pltpu.sync_copy(hw_hbm.at[pl.ds(c * NP + base, CH)], buf)

            @plsc.parallel_loop(0, CH, unroll=4)
            def init_e(e):
                dvec = _splat(dis_v, base + e)
                for g in range(HALF // L):
                    sl = pl.ds(g * L, L)
                    buf[e, sl] = buf[e, sl] * dvec
            pltpu.sync_copy(buf, acc_sh.at[pl.ds(base, CH)])
            return 0
        lax.fori_loop(0, NNCH, init_chunk, 0)
        plsc.subcore_barrier()

        def prep(b):
            @plsc.parallel_loop(0, CH // L, unroll=5)
            def grp(g):
                sl = pl.ds(g * L, L)
                r16 = epk_v[b][0, sl]
                rowo_v[b][sl] = r16 + coff
                d16 = plsc.load_gather(dis_v, [r16])
                e16 = lax.bitcast_convert_type(epk_v[b][2, sl], jnp.float32)
                w_v[b][sl] = d16 * e16
                col_v[b][sl] = epk_v[b][1, sl]

        def issue_gather(b):
            pltpu.async_copy(hw_hbm.at[rowo_v[b]], rows_v[b], sem_g[b])

        def wait_gather(b):
            pltpu.make_async_copy(hw_hbm.at[rowo_v[b]], rows_v[b],
                                  sem_g[b]).wait()

        def issue_scatter(b):
            pltpu.async_copy(rows_v[b], acc_sh.at[col_v[b]], sem_s,
                             add=True)

        def wait_scatter(b):
            pltpu.make_async_copy(rows_v[b], acc_sh.at[col_v[b]],
                                  sem_s).wait()

        def scale(b):
            @plsc.parallel_loop(0, CH, unroll=4)
            def scale_e(e):
                wvec = _splat(w_v[b], e)
                for g in range(HALF // L):
                    sl = pl.ds(g * L, L)
                    rows_v[b][e, sl] = rows_v[b][e, sl] * wvec

        def edge_iter(j, b, first):
            pltpu.make_async_copy(epk_hbm.at[s, j + 1], epk_v[1 - b],
                                  sem_e).wait()
            if not first:
                wait_scatter(1 - b)
            prep(1 - b)
            issue_gather(1 - b)
            pltpu.async_copy(epk_hbm.at[s, j + 2], epk_v[b], sem_e)
            wait_gather(b)
            scale(b)
            issue_scatter(b)

        pltpu.sync_copy(epk_hbm.at[s, 0], epk_v[0])
        prep(0)
        issue_gather(0)
        pltpu.async_copy(epk_hbm.at[s, 1], epk_v[1], sem_e)
        edge_iter(0, 0, True)

        def edge_pair(t, _):
            edge_iter(2 * t + 1, 1, False)
            edge_iter(2 * t + 2, 0, False)
            return 0
        lax.fori_loop(0, (NECH - 1) // 2, edge_pair, 0)
        wait_scatter(0)
        wait_gather(1)
        pltpu.make_async_copy(epk_hbm.at[s, NECH + 1], epk_v[0],
                              sem_e).wait()
        plsc.subcore_barrier()

        def flush_chunk(k, _):
            base = nbase + k * CH
            buf = rows_v[0]

            def do_flush():
                pltpu.sync_copy(acc_sh.at[pl.ds(base, CH)], buf)

                @plsc.parallel_loop(0, CH, unroll=4)
                def flush_e(e):
                    dvec = _splat(dis_v, base + e)
                    for g in range(HALF // L):
                        sl = pl.ds(g * L, L)
                        v = buf[e, sl] * dvec
                        if final:
                            v = 1.0 / (1.0 + jnp.exp(-v))
                        else:
                            v = jnp.maximum(v, 0.0)
                        buf[e, sl] = v
                if final:
                    pltpu.sync_copy(
                        buf, h_out.at[pl.ds(base, CH),
                                      pl.ds(c * HALF, HALF)])
                else:
                    pltpu.sync_copy(
                        buf, h_out.at[pl.ds(c * NP + base, CH)])

            if final:
                @pl.when(base < N)
                def _():
                    do_flush()
            else:
                do_flush()
            return 0
        lax.fori_loop(0, NNCH, flush_chunk, 0)

    return kprop


_sc_deg = _sc_degree()
_sc_prop_mid = _make_sc_propagate(False)
_sc_prop_final = _make_sc_propagate(True)


@jax.jit
def kernel(x, edge_index, edge_weight, W1, b1, W2, b2):
    row = edge_index[0].reshape(NS, NECH, CH)
    col = edge_index[1].reshape(NS, NECH, CH)
    ewb = lax.bitcast_convert_type(
        edge_weight.reshape(NS, NECH, CH), jnp.int32)
    epk = jnp.stack([row, col, ewb], axis=2)
    epk = jnp.pad(epk, ((0, 0), (0, 2), (0, 0), (0, 0)))

    dis = _sc_deg(epk)
    xp = jnp.pad(x, ((0, NP - N), (0, 0)))
    hw1 = _linear_halves_from_full(xp, W1, b1).reshape(NC * NP, HALF)
    h1 = _sc_prop_mid(epk, hw1, dis).reshape(NC, NP, HALF)
    hw2 = _linear_halves_from_halves(h1, W2, b2).reshape(NC * NP, HALF)
    return _sc_prop_final(epk, hw2, dis)


# speedup vs baseline: 11.3641x; 1.0588x over previous
import functools

import jax
import jax.numpy as jnp
from jax import lax
from jax.experimental import pallas as pl
from jax.experimental.pallas import tpu as pltpu
from jax.experimental.pallas import tpu_sc as plsc

N = 10000
E = 160000
D = 256
HALF = 128
NC = 2
NS = 16
L = 16
NP = 10240
NPS = NP // NS
CH = 80
EPS = E // NS
NECH = EPS // CH
NNCH = NPS // CH
RB = 512


def _lin_body(xa_ref, xb_ref, w_ref, b_ref, o_ref):
    x = jnp.concatenate([xa_ref[...].reshape(RB, HALF),
                         xb_ref[...].reshape(RB, HALF)], axis=1)
    w = w_ref[0]
    o = lax.dot_general(x, w, (((1,), (1,)), ((), ())),
                        preferred_element_type=jnp.float32)
    o_ref[0] = o + b_ref[0]


def _linear_halves_from_full(x, W, b):
    w3 = W.reshape(NC, HALF, D)
    b2 = b.reshape(NC, 1, HALF)
    grid = (NC, NP // RB)
    return pl.pallas_call(
        _lin_body,
        grid=grid,
        in_specs=[
            pl.BlockSpec((RB, HALF), lambda c, i: (i, 0)),
            pl.BlockSpec((RB, HALF), lambda c, i: (i, 1)),
            pl.BlockSpec((1, HALF, D), lambda c, i: (c, 0, 0)),
            pl.BlockSpec((1, 1, HALF), lambda c, i: (c, 0, 0)),
        ],
        out_specs=pl.BlockSpec((1, RB, HALF), lambda c, i: (c, i, 0)),
        out_shape=jax.ShapeDtypeStruct((NC, NP, HALF), jnp.float32),
    )(x, x, w3, b2)


def _linear_halves_from_halves(h, W, b):
    w3 = W.reshape(NC, HALF, D)
    b2 = b.reshape(NC, 1, HALF)
    grid = (NC, NP // RB)
    return pl.pallas_call(
        _lin_body,
        grid=grid,
        in_specs=[
            pl.BlockSpec((1, RB, HALF), lambda c, i: (0, i, 0)),
            pl.BlockSpec((1, RB, HALF), lambda c, i: (1, i, 0)),
            pl.BlockSpec((1, HALF, D), lambda c, i: (c, 0, 0)),
            pl.BlockSpec((1, 1, HALF), lambda c, i: (c, 0, 0)),
        ],
        out_specs=pl.BlockSpec((1, RB, HALF), lambda c, i: (c, i, 0)),
        out_shape=jax.ShapeDtypeStruct((NC, NP, HALF), jnp.float32),
    )(h, h, w3, b2)


def _rsqrt16(x):
    i = lax.bitcast_convert_type(x, jnp.int32)
    magic = jnp.full((L,), 0x5F3759DF, jnp.int32)
    y = lax.bitcast_convert_type(magic - lax.shift_right_logical(i, 1),
                                 jnp.float32)
    half = x * (-0.5)
    for _ in range(3):
        y = y * (half * y * y + 1.5)
    return y


def _splat(ref, i):
    return plsc.load_gather(ref, [jnp.full((L,), i, jnp.int32)])


def _sc_degree():
    mesh = plsc.VectorSubcoreMesh(core_axis_name="c", subcore_axis_name="s",
                                  num_cores=NC)
    scratch = dict(
        epk_v=[pltpu.VMEM((3, CH), jnp.int32)] * 2,
        t_v=pltpu.VMEM((NPS,), jnp.float32),
        ones_v=pltpu.VMEM((CH,), jnp.float32),
        deg_sh=pltpu.VMEM_SHARED((NP,), jnp.float32),
        sem_e=pltpu.SemaphoreType.DMA,
    )

    @functools.partial(
        pl.kernel, out_type=jax.ShapeDtypeStruct((NP,), jnp.float32),
        mesh=mesh, scratch_types=scratch,
        compiler_params=pltpu.CompilerParams(needs_layout_passes=False))
    def kdeg(epk_hbm, dis_out, *, epk_v, t_v, ones_v, deg_sh, sem_e):
        c = lax.axis_index("c")
        s = lax.axis_index("s")
        nbase = s * NPS
        zero16 = jnp.zeros((L,), jnp.float32)
        one16 = jnp.ones((L,), jnp.float32)

        def fill(ref, n, vec):
            def st(i, _):
                ref[pl.ds(i * L, L)] = vec
                return 0
            lax.fori_loop(0, n // L, st, 0)

        fill(ones_v, CH, one16)
        fill(t_v, NPS, zero16)
        pltpu.sync_copy(t_v, deg_sh.at[pl.ds(nbase, NPS)])
        plsc.subcore_barrier()

        pltpu.sync_copy(epk_hbm.at[s, 0], epk_v[0])
        pltpu.async_copy(epk_hbm.at[s, 1], epk_v[1], sem_e)

        def deg_iter(j, b):
            pltpu.sync_copy(ones_v, deg_sh.at[epk_v[b].at[1]], add=True)
            pltpu.make_async_copy(epk_hbm.at[s, j + 1], epk_v[1 - b],
                                  sem_e).wait()
            pltpu.async_copy(epk_hbm.at[s, j + 2], epk_v[b], sem_e)

        def deg_pair(t, _):
            deg_iter(2 * t, 0)
            deg_iter(2 * t + 1, 1)
            return 0
        lax.fori_loop(0, (NECH - 1) // 2, deg_pair, 0)
        pltpu.sync_copy(ones_v, deg_sh.at[epk_v[0].at[1]], add=True)
        pltpu.make_async_copy(epk_hbm.at[s, NECH], epk_v[1], sem_e).wait()
        plsc.subcore_barrier()

        pltpu.sync_copy(deg_sh.at[pl.ds(nbase, NPS)], t_v)

        def mk_dis(i, _):
            sl = pl.ds(i * L, L)
            t_v[sl] = _rsqrt16(t_v[sl] + 1.0)
            return 0
        lax.fori_loop(0, NPS // L, mk_dis, 0)

        @pl.when(c == 0)
        def _():
            pltpu.sync_copy(t_v, dis_out.at[pl.ds(nbase, NPS)])

    return kdeg


def _make_sc_propagate(final: bool):
    mesh = plsc.VectorSubcoreMesh(core_axis_name="c", subcore_axis_name="s",
                                  num_cores=NC)
    if final:
        out_type = jax.ShapeDtypeStruct((N, D), jnp.float32)
    else:
        out_type = jax.ShapeDtypeStruct((NC * NP, HALF), jnp.float32)

    scratch = dict(
        epk_v=[pltpu.VMEM((3, CH), jnp.int32)] * 3,
        rowo_v=[pltpu.VMEM((CH,), jnp.int32)] * 3,
        col_v=[pltpu.VMEM((CH,), jnp.int32)] * 3,
        w_v=[pltpu.VMEM((CH,), jnp.float32)] * 3,
        rows_v=[pltpu.VMEM((CH, HALF), jnp.float32)] * 3,
        dis_v=pltpu.VMEM((NP,), jnp.float32),
        acc_sh=pltpu.VMEM_SHARED((NP, HALF), jnp.float32),
        sem_e=pltpu.SemaphoreType.DMA,
        sem_g=[pltpu.SemaphoreType.DMA] * 3,
        sem_s=[pltpu.SemaphoreType.DMA] * 3,
    )

    @functools.partial(
        pl.kernel, out_type=out_type, mesh=mesh, scratch_types=scratch,
        compiler_params=pltpu.CompilerParams(needs_layout_passes=False))
    def kprop(epk_hbm, hw_hbm, dis_hbm, h_out, *, epk_v, rowo_v, col_v,
              w_v, rows_v, dis_v, acc_sh, sem_e, sem_g, sem_s):
        c = lax.axis_index("c")
        s = lax.axis_index("s")
        nbase = s * NPS
        coff = jnp.full((L,), c * NP, jnp.int32)

        pltpu.sync_copy(dis_hbm, dis_v)

        def init_chunk(k, _):
            base = nbase + k * CH
            buf = rows_v[0]
            pltpu.sync_copy(hw_hbm.at[pl.ds(c * NP + base, CH)], buf)

            @plsc.parallel_loop(0, CH, unroll=4)
            def init_e(e):
                dvec = _splat(dis_v, base + e)
                for g in range(HALF // L):
                    sl = pl.ds(g * L, L)
                    buf[e, sl] = buf[e, sl] * dvec
            pltpu.sync_copy(buf, acc_sh.at[pl.ds(base, CH)])
            return 0
        lax.fori_loop(0, NNCH, init_chunk, 0)
        plsc.subcore_barrier()

        def prep(b):
            @plsc.parallel_loop(0, CH // L, unroll=5)
            def grp(g):
                sl = pl.ds(g * L, L)
                r16 = epk_v[b][0, sl]
                rowo_v[b][sl] = r16 + coff
                d16 = plsc.load_gather(dis_v, [r16])
                e16 = lax.bitcast_convert_type(epk_v[b][2, sl], jnp.float32)
                w_v[b][sl] = d16 * e16
                col_v[b][sl] = epk_v[b][1, sl]

        def issue_gather(b):
            pltpu.async_copy(hw_hbm.at[rowo_v[b]], rows_v[b], sem_g[b])

        def wait_gather(b):
            pltpu.make_async_copy(hw_hbm.at[rowo_v[b]], rows_v[b],
                                  sem_g[b]).wait()

        def issue_scatter(b):
            pltpu.async_copy(rows_v[b], acc_sh.at[col_v[b]], sem_s[b],
                             add=True)

        def wait_scatter(b):
            pltpu.make_async_copy(rows_v[b], acc_sh.at[col_v[b]],
                                  sem_s[b]).wait()

        def scale(b):
            @plsc.parallel_loop(0, CH, unroll=4)
            def scale_e(e):
                wvec = _splat(w_v[b], e)
                for g in range(HALF // L):
                    sl = pl.ds(g * L, L)
                    rows_v[b][e, sl] = rows_v[b][e, sl] * wvec

        def edge_iter(j, b, first):
            n = (b + 1) % 3
            p = (b + 2) % 3
            pltpu.make_async_copy(epk_hbm.at[s, j + 1], epk_v[n],
                                  sem_e).wait()
            prep(n)
            issue_gather(n)
            pltpu.async_copy(epk_hbm.at[s, j + 2], epk_v[p], sem_e)
            wait_gather(b)
            scale(b)
            if not first:
                wait_scatter(p)
            issue_scatter(b)

        pltpu.sync_copy(epk_hbm.at[s, 0], epk_v[0])
        prep(0)
        issue_gather(0)
        pltpu.async_copy(epk_hbm.at[s, 1], epk_v[1], sem_e)
        edge_iter(0, 0, True)
        edge_iter(1, 1, False)

        def edge_triple(t, _):
            edge_iter(3 * t + 2, 2, False)
            edge_iter(3 * t + 3, 0, False)
            edge_iter(3 * t + 4, 1, False)
            return 0
        lax.fori_loop(0, (NECH - 2) // 3, edge_triple, 0)
        wait_scatter(1)
        wait_gather(2)
        pltpu.make_async_copy(epk_hbm.at[s, NECH + 1],
                              epk_v[(NECH + 1) % 3], sem_e).wait()
        plsc.subcore_barrier()

        def flush_chunk(k, _):
            base = nbase + k * CH
            buf = rows_v[0]

            def do_flush():
                pltpu.sync_copy(acc_sh.at[pl.ds(base, CH)], buf)

                @plsc.parallel_loop(0, CH, unroll=4)
                def flush_e(e):
                    dvec = _splat(dis_v, base + e)
                    for g in range(HALF // L):
                        sl = pl.ds(g * L, L)
                        v = buf[e, sl] * dvec
                        if final:
                            v = 1.0 / (1.0 + jnp.exp(-v))
                        else:
                            v = jnp.maximum(v, 0.0)
                        buf[e, sl] = v
                if final:
                    pltpu.sync_copy(
                        buf, h_out.at[pl.ds(base, CH),
                                      pl.ds(c * HALF, HALF)])
                else:
                    pltpu.sync_copy(
                        buf, h_out.at[pl.ds(c * NP + base, CH)])

            if final:
                @pl.when(base < N)
                def _():
                    do_flush()
            else:
                do_flush()
            return 0
        lax.fori_loop(0, NNCH, flush_chunk, 0)

    return kprop


_sc_deg = _sc_degree()
_sc_prop_mid = _make_sc_propagate(False)
_sc_prop_final = _make_sc_propagate(True)


@jax.jit
def kernel(x, edge_index, edge_weight, W1, b1, W2, b2):
    row = edge_index[0].reshape(NS, NECH, CH)
    col = edge_index[1].reshape(NS, NECH, CH)
    ewb = lax.bitcast_convert_type(
        edge_weight.reshape(NS, NECH, CH), jnp.int32)
    epk = jnp.stack([row, col, ewb], axis=2)
    epk = jnp.pad(epk, ((0, 0), (0, 2), (0, 0), (0, 0)))

    dis = _sc_deg(epk)
    xp = jnp.pad(x, ((0, NP - N), (0, 0)))
    hw1 = _linear_halves_from_full(xp, W1, b1).reshape(NC * NP, HALF)
    h1 = _sc_prop_mid(epk, hw1, dis).reshape(NC, NP, HALF)
    hw2 = _linear_halves_from_halves(h1, W2, b2).reshape(NC * NP, HALF)
    return _sc_prop_final(epk, hw2, dis)
